# baseline (speedup 1.0000x reference)
import jax
import jax.numpy as jnp
from jax import lax
from jax.experimental import pallas as pl
from jax.experimental.pallas import tpu as pltpu
from jax.experimental.pallas import tpu_sc as plsc

_N = 10000
_E = 320000
_D = 128
_HALF = _D // 2
_NS = 16
_B = 128
_NIT = 160
_EPT = _NIT * _B
_EPAD = _NS * _EPT
_NPAD = 10240
_RPT = _NPAD // _NS
_ZR = 128
_NROW = 4
_NIDX = 8


def _body(xc_hbm, epk_hbm, out_hbm, ebuf, rows, zbuf, shared, xs,
          isem, gsem, ssem):
    c = lax.axis_index("c")
    s = lax.axis_index("s")

    zero = jnp.zeros((16,), jnp.float32)

    def zrow(i, carry):
        for cc in range(_HALF // 16):
            zbuf[i, pl.ds(cc * 16, 16)] = zero
        return carry

    lax.fori_loop(0, _ZR, zrow, 0)
    for k in range(_RPT // _ZR):
        pltpu.sync_copy(zbuf, shared.at[pl.ds(s * _RPT + k * _ZR, _ZR)])
    pltpu.sync_copy(xc_hbm.at[pl.ds(c * _NPAD + s * _RPT, _RPT)],
                    xs.at[pl.ds(s * _RPT, _RPT)])
    plsc.subcore_barrier()

    bbase = s * _NIT

    def issue_idx(it, q):
        pltpu.async_copy(epk_hbm.at[bbase + it], ebuf.at[q], isem.at[q])

    def wait_idx(it, q):
        pltpu.make_async_copy(epk_hbm.at[bbase + it], ebuf.at[q],
                              isem.at[q]).wait()

    def adjust_and_gather(q, p, hbm):
        if hbm:
            for gg in range(_B // 16):
                sl = pl.ds(gg * 16, 16)
                ebuf[q, 0, sl] = ebuf[q, 0, sl] + c * _NPAD
            pltpu.async_copy(xc_hbm.at[ebuf.at[q, 0]], rows.at[p], gsem.at[p])
        else:
            pltpu.async_copy(xs.at[ebuf.at[q, 0]], rows.at[p], gsem.at[p])

    def wait_gather(q, p, hbm):
        if hbm:
            pltpu.make_async_copy(xc_hbm.at[ebuf.at[q, 0]], rows.at[p],
                                  gsem.at[p]).wait()
        else:
            pltpu.make_async_copy(xs.at[ebuf.at[q, 0]], rows.at[p],
                                  gsem.at[p]).wait()

    def issue_scatter(q, p):
        pltpu.async_copy(rows.at[p], shared.at[ebuf.at[q, 1]], ssem.at[p],
                         add=True)

    def wait_scatter(q, p):
        pltpu.make_async_copy(rows.at[p], shared.at[ebuf.at[q, 1]],
                              ssem.at[p]).wait()

    for k in range(4):
        issue_idx(k, k)
    for k in range(2):
        wait_idx(k, k)
        adjust_and_gather(k, k, k % 2 == 1)

    def step(it, jj):
        q = jj % _NIDX
        p = jj % _NROW
        q2 = (jj + 2) % _NIDX
        p2 = (jj + 2) % _NROW

        hbm = jj % 2 == 1

        wait_gather(q, p, hbm)

        @pl.when(it + 2 < _NIT)
        def _():
            wait_idx(it + 2, q2)

            @pl.when(it >= 2)
            def _():
                wait_scatter((jj - 2) % _NIDX, p2)

            adjust_and_gather(q2, p2, hbm)

        def grp(gg, carry2):
            a16 = plsc.bitcast(ebuf[q, 2, pl.ds(gg * 16, 16)], jnp.float32)
            for j in range(16):
                avj = jnp.full((16,), a16[j], jnp.float32)
                for cc in range(_HALF // 16):
                    sl = pl.ds(cc * 16, 16)
                    rows[p, gg * 16 + j, sl] = rows[p, gg * 16 + j, sl] * avj
            return carry2

        lax.fori_loop(0, _B // 16, grp, 0)

        issue_scatter(q, p)

        @pl.when(it + 4 < _NIT)
        def _():
            issue_idx(it + 4, (jj + 4) % _NIDX)

    def outer(g, carry):
        for jj in range(8):
            step(8 * g + jj, jj)
        return carry

    lax.fori_loop(0, _NIT // 8, outer, 0)

    for k in range(4):
        it = _NIT - 4 + k
        wait_scatter(it % _NIDX, it % _NROW)

    plsc.subcore_barrier()
    pltpu.sync_copy(shared.at[pl.ds(s * _RPT, _RPT)],
                    out_hbm.at[pl.ds(c * _NPAD + s * _RPT, _RPT)])


@jax.jit
def _propagate(x_cat, epk):
    mesh = plsc.VectorSubcoreMesh(core_axis_name="c", subcore_axis_name="s",
                                  num_cores=2, num_subcores=_NS)
    k = pl.kernel(
        _body,
        out_type=jax.ShapeDtypeStruct((2 * _NPAD, _HALF), jnp.float32),
        mesh=mesh,
        scratch_types=[
            pltpu.VMEM((_NIDX, 3, _B), jnp.int32),
            pltpu.VMEM((_NROW, _B, _HALF), jnp.float32),
            pltpu.VMEM((_ZR, _HALF), jnp.float32),
            pltpu.VMEM_SHARED((_NPAD, _HALF), jnp.float32),
            pltpu.VMEM_SHARED((_NPAD, _HALF), jnp.float32),
            pltpu.SemaphoreType.DMA((_NIDX,)),
            pltpu.SemaphoreType.DMA((_NROW,)),
            pltpu.SemaphoreType.DMA((_NROW,)),
        ],
        compiler_params=pltpu.CompilerParams(use_tc_tiling_on_sc=False,
                                             needs_layout_passes=False),
    )
    return k(x_cat, epk)


def kernel(x, edge_index, A_values):
    xp = jnp.pad(x, ((0, _NPAD - _N), (0, 0)))
    x_cat = jnp.concatenate([xp[:, :_HALF], xp[:, _HALF:]], axis=0)
    pad = _EPAD - _E
    src = jnp.pad(edge_index[0], (0, pad)).reshape(_NS * _NIT, _B)
    dst = jnp.pad(edge_index[1], (0, pad)).reshape(_NS * _NIT, _B)
    a = lax.bitcast_convert_type(jnp.pad(A_values, (0, pad)),
                                 jnp.int32).reshape(_NS * _NIT, _B)
    epk = jnp.stack([src, dst, a], axis=1)
    out2 = _propagate(x_cat, epk)
    return jnp.concatenate([out2[:_N], out2[_NPAD:_NPAD + _N]], axis=1)


# speedup vs baseline: 1.0352x; 1.0352x over previous
import jax
import jax.numpy as jnp
from jax import lax
from jax.experimental import pallas as pl
from jax.experimental.pallas import tpu as pltpu
from jax.experimental.pallas import tpu_sc as plsc

_N = 10000
_E = 320000
_D = 128
_HALF = _D // 2
_NS = 16
_B = 128
_NIT = 160
_EPT = _NIT * _B
_EPAD = _NS * _EPT
_NPAD = 10240
_RPT = _NPAD // _NS
_ZR = 128
_NROW = 4
_NIDX = 8


def _body(xc_hbm, epk_hbm, out_hbm, ebuf, rows, zbuf, shared, xs,
          isem, gsem, ssem):
    c = lax.axis_index("c")
    s = lax.axis_index("s")

    zero = jnp.zeros((16,), jnp.float32)

    def zrow(i, carry):
        for cc in range(_HALF // 16):
            zbuf[i, pl.ds(cc * 16, 16)] = zero
        return carry

    lax.fori_loop(0, _ZR, zrow, 0)
    for k in range(_RPT // _ZR):
        pltpu.sync_copy(zbuf, shared.at[pl.ds(s * _RPT + k * _ZR, _ZR)])
    pltpu.sync_copy(xc_hbm.at[pl.ds(c * _NPAD + s * _RPT, _RPT)],
                    xs.at[pl.ds(s * _RPT, _RPT)])
    plsc.subcore_barrier()

    bbase = s * _NIT

    def issue_idx(it, q):
        pltpu.async_copy(epk_hbm.at[bbase + it], ebuf.at[q], isem.at[q])

    def wait_idx(it, q):
        pltpu.make_async_copy(epk_hbm.at[bbase + it], ebuf.at[q],
                              isem.at[q]).wait()

    def adjust_and_gather(q, p):
        pltpu.async_copy(xs.at[ebuf.at[q, 0]], rows.at[p], gsem.at[p])

    def wait_gather(q, p):
        pltpu.make_async_copy(xs.at[ebuf.at[q, 0]], rows.at[p],
                              gsem.at[p]).wait()

    def issue_scatter(q, p):
        pltpu.async_copy(rows.at[p], shared.at[ebuf.at[q, 1]], ssem.at[p],
                         add=True)

    def wait_scatter(q, p):
        pltpu.make_async_copy(rows.at[p], shared.at[ebuf.at[q, 1]],
                              ssem.at[p]).wait()

    for k in range(4):
        issue_idx(k, k)
    for k in range(2):
        wait_idx(k, k)
        adjust_and_gather(k, k)

    def step(it, jj):
        q = jj % _NIDX
        p = jj % _NROW
        q2 = (jj + 2) % _NIDX
        p2 = (jj + 2) % _NROW

        wait_gather(q, p)

        @pl.when(it + 2 < _NIT)
        def _():
            wait_idx(it + 2, q2)

            @pl.when(it >= 2)
            def _():
                wait_scatter((jj - 2) % _NIDX, p2)

            adjust_and_gather(q2, p2)

        def grp(gg, carry2):
            a16 = plsc.bitcast(ebuf[q, 2, pl.ds(gg * 16, 16)], jnp.float32)
            for j in range(16):
                avj = jnp.full((16,), a16[j], jnp.float32)
                for cc in range(_HALF // 16):
                    sl = pl.ds(cc * 16, 16)
                    rows[p, gg * 16 + j, sl] = rows[p, gg * 16 + j, sl] * avj
            return carry2

        lax.fori_loop(0, _B // 16, grp, 0)

        issue_scatter(q, p)

        @pl.when(it + 4 < _NIT)
        def _():
            issue_idx(it + 4, (jj + 4) % _NIDX)

    def outer(g, carry):
        for jj in range(8):
            step(8 * g + jj, jj)
        return carry

    lax.fori_loop(0, _NIT // 8, outer, 0)

    for k in range(4):
        it = _NIT - 4 + k
        wait_scatter(it % _NIDX, it % _NROW)

    plsc.subcore_barrier()
    pltpu.sync_copy(shared.at[pl.ds(s * _RPT, _RPT)],
                    out_hbm.at[pl.ds(c * _NPAD + s * _RPT, _RPT)])


@jax.jit
def _propagate(x_cat, epk):
    mesh = plsc.VectorSubcoreMesh(core_axis_name="c", subcore_axis_name="s",
                                  num_cores=2, num_subcores=_NS)
    k = pl.kernel(
        _body,
        out_type=jax.ShapeDtypeStruct((2 * _NPAD, _HALF), jnp.float32),
        mesh=mesh,
        scratch_types=[
            pltpu.VMEM((_NIDX, 3, _B), jnp.int32),
            pltpu.VMEM((_NROW, _B, _HALF), jnp.float32),
            pltpu.VMEM((_ZR, _HALF), jnp.float32),
            pltpu.VMEM_SHARED((_NPAD, _HALF), jnp.float32),
            pltpu.VMEM_SHARED((_NPAD, _HALF), jnp.float32),
            pltpu.SemaphoreType.DMA((_NIDX,)),
            pltpu.SemaphoreType.DMA((_NROW,)),
            pltpu.SemaphoreType.DMA((_NROW,)),
        ],
        compiler_params=pltpu.CompilerParams(use_tc_tiling_on_sc=False,
                                             needs_layout_passes=False),
    )
    return k(x_cat, epk)


def kernel(x, edge_index, A_values):
    xp = jnp.pad(x, ((0, _NPAD - _N), (0, 0)))
    x_cat = jnp.concatenate([xp[:, :_HALF], xp[:, _HALF:]], axis=0)
    pad = _EPAD - _E
    src = jnp.pad(edge_index[0], (0, pad)).reshape(_NS * _NIT, _B)
    dst = jnp.pad(edge_index[1], (0, pad)).reshape(_NS * _NIT, _B)
    a = lax.bitcast_convert_type(jnp.pad(A_values, (0, pad)),
                                 jnp.int32).reshape(_NS * _NIT, _B)
    epk = jnp.stack([src, dst, a], axis=1)
    out2 = _propagate(x_cat, epk)
    return jnp.concatenate([out2[:_N], out2[_NPAD:_NPAD + _N]], axis=1)


# speedup vs baseline: 1.1664x; 1.1268x over previous
import jax
import jax.numpy as jnp
from jax import lax
from jax.experimental import pallas as pl
from jax.experimental.pallas import tpu as pltpu
from jax.experimental.pallas import tpu_sc as plsc

_N = 10000
_E = 320000
_D = 128
_HALF = _D // 2
_NS = 16
_B = 128
_NIT = 160
_EPT = _NIT * _B
_EPAD = _NS * _EPT
_NPAD = 10240
_RPT = _NPAD // _NS
_ZR = 128
_NROW = 4
_NIDX = 8


_XR = 624


def _body(x_hbm, epk_hbm, out_hbm, ebuf, rows, zbuf, shared, xs,
          isem, gsem, ssem):
    c = lax.axis_index("c")
    s = lax.axis_index("s")

    zero = jnp.zeros((16,), jnp.float32)

    def zrow(i, carry):
        for cc in range(_HALF // 16):
            zbuf[i, pl.ds(cc * 16, 16)] = zero
        return carry

    lax.fori_loop(0, _ZR, zrow, 0)
    for k in range(_RPT // _ZR):
        pltpu.sync_copy(zbuf, shared.at[pl.ds(s * _RPT + k * _ZR, _ZR)])
    pltpu.sync_copy(x_hbm.at[pl.ds(s * _XR, _XR), pl.ds(c * _HALF, _HALF)],
                    xs.at[pl.ds(s * _XR, _XR)])

    @pl.when(s == _NS - 1)
    def _():
        pltpu.sync_copy(
            x_hbm.at[pl.ds(_NS * _XR, _N - _NS * _XR),
                     pl.ds(c * _HALF, _HALF)],
            xs.at[pl.ds(_NS * _XR, _N - _NS * _XR)])

    plsc.subcore_barrier()

    bbase = s * _NIT

    def issue_idx(it, q):
        pltpu.async_copy(epk_hbm.at[bbase + it], ebuf.at[q], isem.at[q])

    def wait_idx(it, q):
        pltpu.make_async_copy(epk_hbm.at[bbase + it], ebuf.at[q],
                              isem.at[q]).wait()

    def adjust_and_gather(q, p):
        pltpu.async_copy(xs.at[ebuf.at[q, 0]], rows.at[p], gsem.at[p])

    def wait_gather(q, p):
        pltpu.make_async_copy(xs.at[ebuf.at[q, 0]], rows.at[p],
                              gsem.at[p]).wait()

    def issue_scatter(q, p):
        pltpu.async_copy(rows.at[p], shared.at[ebuf.at[q, 1]], ssem.at[p],
                         add=True)

    def wait_scatter(q, p):
        pltpu.make_async_copy(rows.at[p], shared.at[ebuf.at[q, 1]],
                              ssem.at[p]).wait()

    for k in range(4):
        issue_idx(k, k)
    for k in range(2):
        wait_idx(k, k)
        adjust_and_gather(k, k)

    def step(it, jj):
        q = jj % _NIDX
        p = jj % _NROW
        q2 = (jj + 2) % _NIDX
        p2 = (jj + 2) % _NROW

        wait_gather(q, p)

        @pl.when(it + 2 < _NIT)
        def _():
            wait_idx(it + 2, q2)

            @pl.when(it >= 2)
            def _():
                wait_scatter((jj - 2) % _NIDX, p2)

            adjust_and_gather(q2, p2)

        def grp(gg, carry2):
            a16 = plsc.bitcast(ebuf[q, 2, pl.ds(gg * 16, 16)], jnp.float32)
            for j in range(16):
                avj = jnp.full((16,), a16[j], jnp.float32)
                for cc in range(_HALF // 16):
                    sl = pl.ds(cc * 16, 16)
                    rows[p, gg * 16 + j, sl] = rows[p, gg * 16 + j, sl] * avj
            return carry2

        lax.fori_loop(0, _B // 16, grp, 0)

        issue_scatter(q, p)

        @pl.when(it + 4 < _NIT)
        def _():
            issue_idx(it + 4, (jj + 4) % _NIDX)

    def outer(g, carry):
        for jj in range(8):
            step(8 * g + jj, jj)
        return carry

    lax.fori_loop(0, _NIT // 8, outer, 0)

    for k in range(4):
        it = _NIT - 4 + k
        wait_scatter(it % _NIDX, it % _NROW)

    plsc.subcore_barrier()
    @pl.when(s < _NS - 1)
    def _():
        pltpu.sync_copy(
            shared.at[pl.ds(s * _RPT, _RPT)],
            out_hbm.at[pl.ds(s * _RPT, _RPT), pl.ds(c * _HALF, _HALF)])

    @pl.when(s == _NS - 1)
    def _():
        pltpu.sync_copy(
            shared.at[pl.ds((_NS - 1) * _RPT, _N - (_NS - 1) * _RPT)],
            out_hbm.at[pl.ds((_NS - 1) * _RPT, _N - (_NS - 1) * _RPT),
                       pl.ds(c * _HALF, _HALF)])


@jax.jit
def _propagate(x, epk):
    mesh = plsc.VectorSubcoreMesh(core_axis_name="c", subcore_axis_name="s",
                                  num_cores=2, num_subcores=_NS)
    k = pl.kernel(
        _body,
        out_type=jax.ShapeDtypeStruct((_N, _D), jnp.float32),
        mesh=mesh,
        scratch_types=[
            pltpu.VMEM((_NIDX, 3, _B), jnp.int32),
            pltpu.VMEM((_NROW, _B, _HALF), jnp.float32),
            pltpu.VMEM((_ZR, _HALF), jnp.float32),
            pltpu.VMEM_SHARED((_NPAD, _HALF), jnp.float32),
            pltpu.VMEM_SHARED((_NPAD, _HALF), jnp.float32),
            pltpu.SemaphoreType.DMA((_NIDX,)),
            pltpu.SemaphoreType.DMA((_NROW,)),
            pltpu.SemaphoreType.DMA((_NROW,)),
        ],
        compiler_params=pltpu.CompilerParams(use_tc_tiling_on_sc=False,
                                             needs_layout_passes=False),
    )
    return k(x, epk)


def kernel(x, edge_index, A_values):
    pad = _EPAD - _E
    src = jnp.pad(edge_index[0], (0, pad)).reshape(_NS * _NIT, _B)
    dst = jnp.pad(edge_index[1], (0, pad)).reshape(_NS * _NIT, _B)
    a = lax.bitcast_convert_type(jnp.pad(A_values, (0, pad)),
                                 jnp.int32).reshape(_NS * _NIT, _B)
    epk = jnp.stack([src, dst, a], axis=1)
    return _propagate(x, epk)
